# baseline (speedup 1.0000x reference)
import functools

import jax
import jax.numpy as jnp
from jax import lax
from jax.experimental import pallas as pl
from jax.experimental.pallas import tpu as pltpu
from jax.experimental.pallas import tpu_sc as plsc

_B, _C, _NPC, _D = 64, 256, 64, 64
_NC, _NS = 2, 16
_NW = _NC * _NS
_BPW = _B // _NW
_CBLK = 4
_NCHUNK = _C // _CBLK
_L = 16


def _sqrt16(x):
  i = plsc.bitcast(x, jnp.int32)
  y = plsc.bitcast((i >> 1) + jnp.int32(0x1FBD1DF5), jnp.float32)
  y = 0.5 * (y + x / y)
  y = 0.5 * (y + x / y)
  return y


def _build():
  mesh = plsc.VectorSubcoreMesh(
      core_axis_name="core", subcore_axis_name="sub",
      num_cores=_NC, num_subcores=_NS)

  @functools.partial(
      pl.kernel,
      out_type=jax.ShapeDtypeStruct((_B, _C), jnp.float32),
      mesh=mesh,
      compiler_params=pltpu.CompilerParams(needs_layout_passes=False),
      scratch_types=[
          pltpu.VMEM((2, _CBLK, _NPC, _D), jnp.float32),
          pltpu.VMEM((_D,), jnp.float32),
          pltpu.VMEM((_L,), jnp.float32),
          pltpu.VMEM((_NPC,), jnp.float32),
          pltpu.VMEM((_C,), jnp.float32),
          pltpu.SemaphoreType.DMA,
          pltpu.SemaphoreType.DMA,
      ],
  )
  def ker(probes_hbm, emb_hbm, kw_hbm, out_hbm,
          ebuf, pbuf, kwbuf, d2buf, orow, sem0, sem1):
    wid = lax.axis_index("core") * _NS + lax.axis_index("sub")
    pltpu.sync_copy(kw_hbm, kwbuf)
    neg_inv_kw = kwbuf[...]
    lane = lax.iota(jnp.int32, _L)
    last_mask = lane == (_L - 1)
    sems = (sem0, sem1)

    for bi in range(_BPW):
      b = wid * _BPW + bi
      pltpu.sync_copy(probes_hbm.at[b], pbuf)
      pvecs = [pbuf[pl.ds(k * _L, _L)] for k in range(_D // _L)]

      pltpu.async_copy(emb_hbm.at[b, pl.ds(0, _CBLK)], ebuf.at[0], sem0)
      pltpu.async_copy(emb_hbm.at[b, pl.ds(_CBLK, _CBLK)], ebuf.at[1], sem1)

      def compute_chunk(buf_idx, chunk, b=b, pvecs=pvecs):
        sem = sems[buf_idx]
        pltpu.make_async_copy(
            emb_hbm.at[b, pl.ds(chunk * _CBLK, _CBLK)],
            ebuf.at[buf_idx], sem).wait()

        def class_body(cc, carry):
          @plsc.parallel_loop(0, _NPC, unroll=8)
          def _exemplar(j):
            acc = None
            for k in range(_D // _L):
              e = ebuf[buf_idx, cc, j, pl.ds(k * _L, _L)]
              dfr = pvecs[k] - e + 1e-6
              sq = dfr * dfr
              acc = sq if acc is None else acc + sq
            cs = plsc.cumsum(acc)
            plsc.store_scatter(
                d2buf, [jnp.full((_L,), j, jnp.int32)], cs, mask=last_mask)

          accv = None
          for g in range(_NPC // _L):
            d2 = d2buf[pl.ds(g * _L, _L)]
            act = jnp.exp(_sqrt16(d2) * neg_inv_kw)
            accv = act if accv is None else accv + act
          mean_v = plsc.cumsum(accv) * (1.0 / _NPC)
          cidx = chunk * _CBLK + cc
          plsc.store_scatter(
              orow, [jnp.full((_L,), cidx, jnp.int32)], mean_v, mask=last_mask)
          return carry

        lax.fori_loop(0, _CBLK, class_body, 0)

      def pair_body(t, carry, b=b, compute_chunk=compute_chunk):
        c0 = 2 * t
        compute_chunk(0, c0)

        @pl.when(c0 + 2 < _NCHUNK)
        def _():
          pltpu.async_copy(
              emb_hbm.at[b, pl.ds((c0 + 2) * _CBLK, _CBLK)], ebuf.at[0], sem0)

        compute_chunk(1, c0 + 1)

        @pl.when(c0 + 3 < _NCHUNK)
        def _():
          pltpu.async_copy(
              emb_hbm.at[b, pl.ds((c0 + 3) * _CBLK, _CBLK)], ebuf.at[1], sem1)

        return carry

      lax.fori_loop(0, _NCHUNK // 2, pair_body, 0)
      pltpu.sync_copy(orow, out_hbm.at[b])

  return ker


_KER = _build()


def kernel(probes, emb_mats, kernel_width):
  neg_inv_kw = jnp.broadcast_to(
      (-1.0 / kernel_width[0]).astype(jnp.float32), (_L,))
  return _KER(probes, emb_mats, neg_inv_kw)
